# baseline (speedup 1.0000x reference)
import functools
import math

import jax
import jax.numpy as jnp
from jax import lax
from jax.experimental import pallas as pl
from jax.experimental.pallas import tpu as pltpu
from jax.experimental.pallas import tpu_sc as plsc

N = 10000
NP = 10240
E = 320000
H = 128
NITER = 10
NC = 2
NS = 16
NW = NC * NS
EPW = E // NW
C = 40
K = EPW // C
W = 50
RPT = NP // NS

_mesh = plsc.VectorSubcoreMesh(core_axis_name="c", subcore_axis_name="s")


def _fill(buf, value, width):
    val = jnp.full((16,), value, jnp.float32)

    def row(r, carry):
        for t in range(width // 16):
            buf[r, pl.ds(t * 16, 16)] = val
        return carry

    lax.fori_loop(0, C, row, 0)


def _zero_slice(buf, acc_s, s, width):
    _fill(buf, 0.0, width)

    def cp(i, carry):
        pltpu.sync_copy(buf, acc_s.at[pl.ds(s * RPT + i * C, C)])
        return carry

    lax.fori_loop(0, RPT // C, cp, 0)


@functools.partial(
    pl.kernel,
    out_type=jax.ShapeDtypeStruct((NC * NP, H), jnp.float32),
    mesh=_mesh,
    scratch_types=[
        pltpu.VMEM((W, C), jnp.int32),
        pltpu.VMEM((W, C), jnp.int32),
        pltpu.VMEM((C, H), jnp.float32),
        pltpu.VMEM((C, H), jnp.float32),
        pltpu.VMEM((C, H), jnp.float32),
        pltpu.VMEM((C, H), jnp.float32),
        pltpu.VMEM((C, H), jnp.float32),
        pltpu.SemaphoreType.DMA,
        pltpu.SemaphoreType.DMA,
        pltpu.SemaphoreType.DMA,
        pltpu.SemaphoreType.DMA,
        pltpu.SemaphoreType.DMA,
        pltpu.VMEM_SHARED((NP, H), jnp.float32),
    ],
)
def _edge_kernel(qh, src3, dst3, pout, srcv, dstv, r0, r1, r2, r3, r4,
                 s0, s1, s2, s3, s4, acc_s):
    c = lax.axis_index("c")
    s = lax.axis_index("s")
    w = c * NS + s
    sl = pl.ds(s * RPT, RPT)
    _zero_slice(r0, acc_s, s, H)
    plsc.subcore_barrier()

    D = 5
    bufs = (r0, r1, r2, r3, r4)
    sems = (s0, s1, s2, s3, s4)

    def issue(k, t):
        pltpu.async_copy(qh.at[srcv.at[k]], bufs[t], sems[t])

    def waitb(t):
        pltpu.make_async_copy(qh.at[srcv.at[0]], bufs[t], sems[t]).wait()

    def scat(k, t):
        pltpu.sync_copy(bufs[t], acc_s.at[dstv.at[k]], add=True)

    def window(wi, carry):
        pltpu.sync_copy(src3.at[w, wi], srcv)
        pltpu.sync_copy(dst3.at[w, wi], dstv)
        for t in range(D - 1):
            issue(t, t)

        def group(j, carry2):
            k0 = D * j
            for t in range(D):
                waitb(t)
                issue(k0 + t + D - 1, (t + D - 1) % D)
                scat(k0 + t, t)
            return carry2

        lax.fori_loop(0, (W - D) // D, group, 0)
        b = W - D
        for i in range(D):
            t = (b + i) % D
            waitb(t)
            if b + i + D - 1 < W:
                issue(b + i + D - 1, (b + i + D - 1) % D)
            scat(b + i, t)
        return carry

    lax.fori_loop(0, K // W, window, 0)
    plsc.subcore_barrier()
    pltpu.sync_copy(acc_s.at[sl], pout.at[pl.ds(c * NP + s * RPT, RPT)])


DH = H

@functools.partial(
    pl.kernel,
    out_type=jax.ShapeDtypeStruct((NC * NP, DH), jnp.float32),
    mesh=_mesh,
    scratch_types=[
        pltpu.VMEM((W, C), jnp.int32),
        pltpu.VMEM((C, DH), jnp.float32),
        pltpu.VMEM_SHARED((NP, DH), jnp.float32),
    ],
)
def _deg_kernel(dst3, pout, dstv, ones_v, acc_s):
    c = lax.axis_index("c")
    s = lax.axis_index("s")
    w = c * NS + s
    sl = pl.ds(s * RPT, RPT)
    _zero_slice(ones_v, acc_s, s, DH)
    _fill(ones_v, 1.0, DH)
    plsc.subcore_barrier()

    def window(wi, carry):
        pltpu.sync_copy(dst3.at[w, wi], dstv)

        def chunk(k, carry2):
            pltpu.sync_copy(ones_v, acc_s.at[dstv.at[k]], add=True)
            return carry2

        lax.fori_loop(0, W, chunk, 0)
        return carry

    lax.fori_loop(0, K // W, window, 0)
    plsc.subcore_barrier()
    pltpu.sync_copy(acc_s.at[sl], pout.at[pl.ds(c * NP + s * RPT, RPT)])


_BN = 1024
_GN = NP // _BN


def _embedprep_body(h_ref, emb_ref, w0_ref, b0_ref, w1_ref, b1_ref,
                    d0_ref, d1_ref, q_ref, dinv_ref, rdinv_ref):
    hcol = h_ref[...]
    ids = lax.broadcasted_iota(jnp.int32, (_BN, 32), 1)
    oh = (hcol == ids).astype(jnp.float32)
    x = jnp.dot(oh, emb_ref[...], preferred_element_type=jnp.float32)
    x = jnp.maximum(x @ w0_ref[...] + b0_ref[...], 0.0)
    x = x @ w1_ref[...] + b1_ref[...]
    deg = d0_ref[...] + d1_ref[...] + 1.0
    dinv = lax.rsqrt(deg)
    q_ref[...] = dinv * x
    dinv_ref[...] = dinv
    rdinv_ref[...] = jnp.sqrt(deg)


def _embedprep(h2, emb32, W0, b0, W1, b1, d0, d1):
    bs = lambda shape: pl.BlockSpec(shape, lambda i: (i, 0))
    full = lambda shape: pl.BlockSpec(shape, lambda i: (0, 0))
    return pl.pallas_call(
        _embedprep_body,
        grid=(_GN,),
        in_specs=[bs((_BN, 1)), full((32, H)), full((H, H)), full((1, H)),
                  full((H, H)), full((1, H)), bs((_BN, 1)), bs((_BN, 1))],
        out_specs=(bs((_BN, H)), bs((_BN, 1)), bs((_BN, 1))),
        out_shape=(jax.ShapeDtypeStruct((NP, H), jnp.float32),
                   jax.ShapeDtypeStruct((NP, 1), jnp.float32),
                   jax.ShapeDtypeStruct((NP, 1), jnp.float32)),
    )(h2, emb32, W0, b0, W1, b1, d0, d1)


def _node_body(p0_ref, p1_ref, q_ref, dinv_ref, rdinv_ref, sumh_ref,
               steps_ref, cont_ref, acc_ref, hw_ref, hb_ref,
               qn_ref, sumh2_ref, steps2_ref, cont2_ref, acc2_ref):
    q = q_ref[...]
    dinv = dinv_ref[...]
    prop = q * rdinv_ref[...]
    pn = dinv * (p0_ref[...] + p1_ref[...] + q)
    z = jnp.dot(pn, hw_ref[...], preferred_element_type=jnp.float32)
    hp = jax.nn.sigmoid(z + hb_ref[...])
    sumh = sumh_ref[...]
    cont = cont_ref[...]
    pm = (sumh + hp) < 0.99
    pf = pm.astype(jnp.float32) * cont
    steps2 = steps_ref[...] + pf
    sumh2 = sumh + pf * hp
    cond = pm & (steps2 < float(NITER))
    p = jnp.where(cond, sumh2, 1.0 - sumh2)
    acc2_ref[...] = acc_ref[...] + (pn * p + prop * (1.0 - p)) * cont
    qn_ref[...] = dinv * pn
    sumh2_ref[...] = sumh2
    steps2_ref[...] = steps2
    cont2_ref[...] = pf


def _node_step(p0, p1, q, dinv, rdinv, sumh, steps, cont, acc, hw, hb2):
    bs = lambda shape: pl.BlockSpec(shape, lambda i: (i, 0))
    full = lambda shape: pl.BlockSpec(shape, lambda i: (0, 0))
    return pl.pallas_call(
        _node_body,
        grid=(_GN,),
        in_specs=[bs((_BN, H)), bs((_BN, H)), bs((_BN, H)), bs((_BN, 1)),
                  bs((_BN, 1)), bs((_BN, 1)), bs((_BN, 1)), bs((_BN, 1)),
                  bs((_BN, H)), full((H, 1)), full((1, 1))],
        out_specs=(bs((_BN, H)), bs((_BN, 1)), bs((_BN, 1)),
                   bs((_BN, 1)), bs((_BN, H))),
        out_shape=(jax.ShapeDtypeStruct((NP, H), jnp.float32),
                   jax.ShapeDtypeStruct((NP, 1), jnp.float32),
                   jax.ShapeDtypeStruct((NP, 1), jnp.float32),
                   jax.ShapeDtypeStruct((NP, 1), jnp.float32),
                   jax.ShapeDtypeStruct((NP, H), jnp.float32)),
    )(p0, p1, q, dinv, rdinv, sumh, steps, cont, acc, hw, hb2)


def _readout_body(acc_ref, steps_ref, sumh_ref, wr0_ref, br0_ref, wr1_ref,
                  br1_ref, wr2_ref, br2_ref, y_ref, rem_ref, hsum_ref):
    i = pl.program_id(0)

    @pl.when(i == 0)
    def _():
        hsum_ref[...] = jnp.zeros_like(hsum_ref)

    ridx = i * _BN + lax.broadcasted_iota(jnp.int32, (_BN, 1), 0)
    m = (ridx < N).astype(jnp.float32)
    hout = acc_ref[...] / steps_ref[...] * m
    hsum_ref[...] += jnp.sum(hout, axis=0, keepdims=True)
    rem_ref[...] = 1.0 - sumh_ref[...]

    @pl.when(i == _GN - 1)
    def _():
        hg = hsum_ref[...] / float(N)
        y1 = jnp.maximum(hg @ wr0_ref[...] + br0_ref[...], 0.0)
        y2 = jnp.maximum(y1 @ wr1_ref[...] + br1_ref[...], 0.0)
        y_ref[...] = y2 @ wr2_ref[...] + br2_ref[...]


def _readout(acc, steps, sumh, Wr0, br0, Wr1, br1, Wr2, br2):
    bs = lambda shape: pl.BlockSpec(shape, lambda i: (i, 0))
    full = lambda shape: pl.BlockSpec(shape, lambda i: (0, 0))
    return pl.pallas_call(
        _readout_body,
        grid=(_GN,),
        in_specs=[bs((_BN, H)), bs((_BN, 1)), bs((_BN, 1)),
                  full((H, H // 2)), full((1, H // 2)),
                  full((H // 2, H // 4)), full((1, H // 4)),
                  full((H // 4, 1)), full((1, 1))],
        out_specs=(full((1, 1)), bs((_BN, 1))),
        out_shape=(jax.ShapeDtypeStruct((1, 1), jnp.float32),
                   jax.ShapeDtypeStruct((NP, 1), jnp.float32)),
        scratch_shapes=[pltpu.VMEM((1, H), jnp.float32)],
    )(acc, steps, sumh, Wr0, br0, Wr1, br1, Wr2, br2)


def kernel(g, h, e, snorm_n, snorm_e, emb, W0, b0, W1, b1, halt_w, halt_b,
           Wr0, br0, Wr1, br1, Wr2, br2):
    src3 = g[0].reshape(NW, K // W, W, C)
    dst3 = g[1].reshape(NW, K // W, W, C)

    degw = _deg_kernel(dst3)
    d0 = degw[:NP, :1]
    d1 = degw[NP:, :1]

    h2 = jnp.concatenate([h, jnp.zeros((NP - N,), h.dtype)]).reshape(NP, 1)
    emb32 = jnp.concatenate(
        [emb, jnp.zeros((32 - emb.shape[0], H), jnp.float32)], axis=0)
    q, dinv, rdinv = _embedprep(h2, emb32, W0, b0.reshape(1, H),
                                W1, b1.reshape(1, H), d0, d1)

    sumh = jnp.zeros((NP, 1), jnp.float32)
    steps = jnp.ones((NP, 1), jnp.float32)
    cont = jnp.ones((NP, 1), jnp.float32)
    acc = jnp.zeros((NP, H), jnp.float32)
    hb2 = halt_b.reshape(1, 1)

    for _ in range(NITER):
        pcat = _edge_kernel(q, src3, dst3)
        p0, p1 = pcat[:NP], pcat[NP:]
        q, sumh, steps, cont, acc = _node_step(
            p0, p1, q, dinv, rdinv, sumh, steps, cont, acc, halt_w, hb2)

    y, rem = _readout(acc, steps, sumh, Wr0, br0.reshape(1, H // 2),
                      Wr1, br1.reshape(1, H // 4), Wr2, br2.reshape(1, 1))
    return y, steps[:N, 0], rem[:N, 0]


# speedup vs baseline: 1.0084x; 1.0084x over previous
import functools
import math

import jax
import jax.numpy as jnp
from jax import lax
from jax.experimental import pallas as pl
from jax.experimental.pallas import tpu as pltpu
from jax.experimental.pallas import tpu_sc as plsc

N = 10000
NP = 10240
E = 320000
H = 128
NITER = 10
NC = 2
NS = 16
NW = NC * NS
EPW = E // NW
C = 40
K = EPW // C
W = 50
RPT = NP // NS

_mesh = plsc.VectorSubcoreMesh(core_axis_name="c", subcore_axis_name="s")


def _fill(buf, value, width):
    val = jnp.full((16,), value, jnp.float32)

    def row(r, carry):
        for t in range(width // 16):
            buf[r, pl.ds(t * 16, 16)] = val
        return carry

    lax.fori_loop(0, C, row, 0)


def _zero_slice(buf, acc_s, s, width):
    _fill(buf, 0.0, width)

    def cp(i, carry):
        pltpu.sync_copy(buf, acc_s.at[pl.ds(s * RPT + i * C, C)])
        return carry

    lax.fori_loop(0, RPT // C, cp, 0)


@functools.partial(
    pl.kernel,
    out_type=jax.ShapeDtypeStruct((NC * NP, H), jnp.float32),
    mesh=_mesh,
    scratch_types=[
        pltpu.VMEM((W, C), jnp.int32),
        pltpu.VMEM((W, C), jnp.int32),
        pltpu.VMEM((C, H), jnp.float32),
        pltpu.VMEM((C, H), jnp.float32),
        pltpu.VMEM((C, H), jnp.float32),
        pltpu.VMEM((C, H), jnp.float32),
        pltpu.VMEM((C, H), jnp.float32),
        pltpu.SemaphoreType.DMA,
        pltpu.SemaphoreType.DMA,
        pltpu.SemaphoreType.DMA,
        pltpu.SemaphoreType.DMA,
        pltpu.SemaphoreType.DMA,
        pltpu.VMEM_SHARED((NP, H), jnp.float32),
    ],
)
def _edge_kernel(qh, src3, dst3, pout, srcv, dstv, r0, r1, r2, r3, r4,
                 s0, s1, s2, s3, s4, acc_s):
    c = lax.axis_index("c")
    s = lax.axis_index("s")
    w = c * NS + s
    sl = pl.ds(s * RPT, RPT)

    D = 5
    bufs = (r0, r1, r2, r3, r4)
    sems = (s0, s1, s2, s3, s4)

    def issue(k, t):
        pltpu.async_copy(qh.at[srcv.at[k]], bufs[t], sems[t])

    def waitb(t):
        pltpu.make_async_copy(qh.at[srcv.at[0]], bufs[t], sems[t]).wait()

    def scat(k, t):
        pltpu.sync_copy(bufs[t], acc_s.at[dstv.at[k]], add=True)

    def stage(wi):
        pltpu.sync_copy(src3.at[w, wi], srcv)
        pltpu.sync_copy(dst3.at[w, wi], dstv)

    def ring():
        def group(j, carry2):
            k0 = D * j
            for t in range(D):
                waitb(t)
                issue(k0 + t + D - 1, (t + D - 1) % D)
                scat(k0 + t, t)
            return carry2

        lax.fori_loop(0, (W - D) // D, group, 0)
        b = W - D
        for i in range(D):
            t = (b + i) % D
            waitb(t)
            if b + i + D - 1 < W:
                issue(b + i + D - 1, (b + i + D - 1) % D)
            scat(b + i, t)

    stage(0)
    for t in range(D - 1):
        issue(t, t)
    _zero_slice(r4, acc_s, s, H)
    plsc.subcore_barrier()
    ring()

    def window(wi, carry):
        stage(wi)
        for t in range(D - 1):
            issue(t, t)
        ring()
        return carry

    lax.fori_loop(1, K // W, window, 0)
    plsc.subcore_barrier()
    pltpu.sync_copy(acc_s.at[sl], pout.at[pl.ds(c * NP + s * RPT, RPT)])


DH = H

@functools.partial(
    pl.kernel,
    out_type=jax.ShapeDtypeStruct((NC * NP, DH), jnp.float32),
    mesh=_mesh,
    scratch_types=[
        pltpu.VMEM((W, C), jnp.int32),
        pltpu.VMEM((C, DH), jnp.float32),
        pltpu.VMEM_SHARED((NP, DH), jnp.float32),
    ],
)
def _deg_kernel(dst3, pout, dstv, ones_v, acc_s):
    c = lax.axis_index("c")
    s = lax.axis_index("s")
    w = c * NS + s
    sl = pl.ds(s * RPT, RPT)
    _zero_slice(ones_v, acc_s, s, DH)
    _fill(ones_v, 1.0, DH)
    plsc.subcore_barrier()

    def window(wi, carry):
        pltpu.sync_copy(dst3.at[w, wi], dstv)

        def chunk(k, carry2):
            pltpu.sync_copy(ones_v, acc_s.at[dstv.at[k]], add=True)
            return carry2

        lax.fori_loop(0, W, chunk, 0)
        return carry

    lax.fori_loop(0, K // W, window, 0)
    plsc.subcore_barrier()
    pltpu.sync_copy(acc_s.at[sl], pout.at[pl.ds(c * NP + s * RPT, RPT)])


_BN = 1024
_GN = NP // _BN


def _embedprep_body(h_ref, emb_ref, w0_ref, b0_ref, w1_ref, b1_ref,
                    d0_ref, d1_ref, q_ref, dinv_ref, rdinv_ref):
    hcol = h_ref[...]
    ids = lax.broadcasted_iota(jnp.int32, (_BN, 32), 1)
    oh = (hcol == ids).astype(jnp.float32)
    x = jnp.dot(oh, emb_ref[...], preferred_element_type=jnp.float32)
    x = jnp.maximum(x @ w0_ref[...] + b0_ref[...], 0.0)
    x = x @ w1_ref[...] + b1_ref[...]
    deg = d0_ref[...] + d1_ref[...] + 1.0
    dinv = lax.rsqrt(deg)
    q_ref[...] = dinv * x
    dinv_ref[...] = dinv
    rdinv_ref[...] = jnp.sqrt(deg)


def _embedprep(h2, emb32, W0, b0, W1, b1, d0, d1):
    bs = lambda shape: pl.BlockSpec(shape, lambda i: (i, 0))
    full = lambda shape: pl.BlockSpec(shape, lambda i: (0, 0))
    return pl.pallas_call(
        _embedprep_body,
        grid=(_GN,),
        in_specs=[bs((_BN, 1)), full((32, H)), full((H, H)), full((1, H)),
                  full((H, H)), full((1, H)), bs((_BN, 1)), bs((_BN, 1))],
        out_specs=(bs((_BN, H)), bs((_BN, 1)), bs((_BN, 1))),
        out_shape=(jax.ShapeDtypeStruct((NP, H), jnp.float32),
                   jax.ShapeDtypeStruct((NP, 1), jnp.float32),
                   jax.ShapeDtypeStruct((NP, 1), jnp.float32)),
    )(h2, emb32, W0, b0, W1, b1, d0, d1)


def _node_body(p0_ref, p1_ref, q_ref, dinv_ref, rdinv_ref, sumh_ref,
               steps_ref, cont_ref, acc_ref, hw_ref, hb_ref,
               qn_ref, sumh2_ref, steps2_ref, cont2_ref, acc2_ref):
    q = q_ref[...]
    dinv = dinv_ref[...]
    prop = q * rdinv_ref[...]
    pn = dinv * (p0_ref[...] + p1_ref[...] + q)
    z = jnp.dot(pn, hw_ref[...], preferred_element_type=jnp.float32)
    hp = jax.nn.sigmoid(z + hb_ref[...])
    sumh = sumh_ref[...]
    cont = cont_ref[...]
    pm = (sumh + hp) < 0.99
    pf = pm.astype(jnp.float32) * cont
    steps2 = steps_ref[...] + pf
    sumh2 = sumh + pf * hp
    cond = pm & (steps2 < float(NITER))
    p = jnp.where(cond, sumh2, 1.0 - sumh2)
    acc2_ref[...] = acc_ref[...] + (pn * p + prop * (1.0 - p)) * cont
    qn_ref[...] = dinv * pn
    sumh2_ref[...] = sumh2
    steps2_ref[...] = steps2
    cont2_ref[...] = pf


def _node_step(p0, p1, q, dinv, rdinv, sumh, steps, cont, acc, hw, hb2):
    bs = lambda shape: pl.BlockSpec(shape, lambda i: (i, 0))
    full = lambda shape: pl.BlockSpec(shape, lambda i: (0, 0))
    return pl.pallas_call(
        _node_body,
        grid=(_GN,),
        in_specs=[bs((_BN, H)), bs((_BN, H)), bs((_BN, H)), bs((_BN, 1)),
                  bs((_BN, 1)), bs((_BN, 1)), bs((_BN, 1)), bs((_BN, 1)),
                  bs((_BN, H)), full((H, 1)), full((1, 1))],
        out_specs=(bs((_BN, H)), bs((_BN, 1)), bs((_BN, 1)),
                   bs((_BN, 1)), bs((_BN, H))),
        out_shape=(jax.ShapeDtypeStruct((NP, H), jnp.float32),
                   jax.ShapeDtypeStruct((NP, 1), jnp.float32),
                   jax.ShapeDtypeStruct((NP, 1), jnp.float32),
                   jax.ShapeDtypeStruct((NP, 1), jnp.float32),
                   jax.ShapeDtypeStruct((NP, H), jnp.float32)),
    )(p0, p1, q, dinv, rdinv, sumh, steps, cont, acc, hw, hb2)


def _readout_body(acc_ref, steps_ref, sumh_ref, wr0_ref, br0_ref, wr1_ref,
                  br1_ref, wr2_ref, br2_ref, y_ref, rem_ref, hsum_ref):
    i = pl.program_id(0)

    @pl.when(i == 0)
    def _():
        hsum_ref[...] = jnp.zeros_like(hsum_ref)

    ridx = i * _BN + lax.broadcasted_iota(jnp.int32, (_BN, 1), 0)
    m = (ridx < N).astype(jnp.float32)
    hout = acc_ref[...] / steps_ref[...] * m
    hsum_ref[...] += jnp.sum(hout, axis=0, keepdims=True)
    rem_ref[...] = 1.0 - sumh_ref[...]

    @pl.when(i == _GN - 1)
    def _():
        hg = hsum_ref[...] / float(N)
        y1 = jnp.maximum(hg @ wr0_ref[...] + br0_ref[...], 0.0)
        y2 = jnp.maximum(y1 @ wr1_ref[...] + br1_ref[...], 0.0)
        y_ref[...] = y2 @ wr2_ref[...] + br2_ref[...]


def _readout(acc, steps, sumh, Wr0, br0, Wr1, br1, Wr2, br2):
    bs = lambda shape: pl.BlockSpec(shape, lambda i: (i, 0))
    full = lambda shape: pl.BlockSpec(shape, lambda i: (0, 0))
    return pl.pallas_call(
        _readout_body,
        grid=(_GN,),
        in_specs=[bs((_BN, H)), bs((_BN, 1)), bs((_BN, 1)),
                  full((H, H // 2)), full((1, H // 2)),
                  full((H // 2, H // 4)), full((1, H // 4)),
                  full((H // 4, 1)), full((1, 1))],
        out_specs=(full((1, 1)), bs((_BN, 1))),
        out_shape=(jax.ShapeDtypeStruct((1, 1), jnp.float32),
                   jax.ShapeDtypeStruct((NP, 1), jnp.float32)),
        scratch_shapes=[pltpu.VMEM((1, H), jnp.float32)],
    )(acc, steps, sumh, Wr0, br0, Wr1, br1, Wr2, br2)


def kernel(g, h, e, snorm_n, snorm_e, emb, W0, b0, W1, b1, halt_w, halt_b,
           Wr0, br0, Wr1, br1, Wr2, br2):
    src3 = g[0].reshape(NW, K // W, W, C)
    dst3 = g[1].reshape(NW, K // W, W, C)

    degw = _deg_kernel(dst3)
    d0 = degw[:NP, :1]
    d1 = degw[NP:, :1]

    h2 = jnp.concatenate([h, jnp.zeros((NP - N,), h.dtype)]).reshape(NP, 1)
    emb32 = jnp.concatenate(
        [emb, jnp.zeros((32 - emb.shape[0], H), jnp.float32)], axis=0)
    q, dinv, rdinv = _embedprep(h2, emb32, W0, b0.reshape(1, H),
                                W1, b1.reshape(1, H), d0, d1)

    sumh = jnp.zeros((NP, 1), jnp.float32)
    steps = jnp.ones((NP, 1), jnp.float32)
    cont = jnp.ones((NP, 1), jnp.float32)
    acc = jnp.zeros((NP, H), jnp.float32)
    hb2 = halt_b.reshape(1, 1)

    for _ in range(NITER):
        pcat = _edge_kernel(q, src3, dst3)
        p0, p1 = pcat[:NP], pcat[NP:]
        q, sumh, steps, cont, acc = _node_step(
            p0, p1, q, dinv, rdinv, sumh, steps, cont, acc, halt_w, hb2)

    y, rem = _readout(acc, steps, sumh, Wr0, br0.reshape(1, H // 2),
                      Wr1, br1.reshape(1, H // 4), Wr2, br2.reshape(1, 1))
    return y, steps[:N, 0], rem[:N, 0]


# speedup vs baseline: 1.0092x; 1.0007x over previous
import functools
import math

import jax
import jax.numpy as jnp
from jax import lax
from jax.experimental import pallas as pl
from jax.experimental.pallas import tpu as pltpu
from jax.experimental.pallas import tpu_sc as plsc

N = 10000
NP = 10240
E = 320000
H = 128
NITER = 10
NC = 2
NS = 16
NW = NC * NS
EPW = E // NW
C = 40
K = EPW // C
W = 50
RPT = NP // NS

_mesh = plsc.VectorSubcoreMesh(core_axis_name="c", subcore_axis_name="s")


def _fill(buf, value, width):
    val = jnp.full((16,), value, jnp.float32)

    def row(r, carry):
        for t in range(width // 16):
            buf[r, pl.ds(t * 16, 16)] = val
        return carry

    lax.fori_loop(0, C, row, 0)


def _zero_slice(buf, acc_s, s, width):
    _fill(buf, 0.0, width)

    def cp(i, carry):
        pltpu.sync_copy(buf, acc_s.at[pl.ds(s * RPT + i * C, C)])
        return carry

    lax.fori_loop(0, RPT // C, cp, 0)


@functools.partial(
    pl.kernel,
    out_type=jax.ShapeDtypeStruct((NC * NP, H), jnp.float32),
    mesh=_mesh,
    scratch_types=[
        pltpu.VMEM((W, C), jnp.int32),
        pltpu.VMEM((W, C), jnp.int32),
        pltpu.VMEM((C, H), jnp.float32),
        pltpu.VMEM((C, H), jnp.float32),
        pltpu.VMEM((C, H), jnp.float32),
        pltpu.VMEM((C, H), jnp.float32),
        pltpu.VMEM((C, H), jnp.float32),
        pltpu.SemaphoreType.DMA,
        pltpu.SemaphoreType.DMA,
        pltpu.SemaphoreType.DMA,
        pltpu.SemaphoreType.DMA,
        pltpu.SemaphoreType.DMA,
        pltpu.VMEM_SHARED((NP, H), jnp.float32),
    ],
)
def _edge_kernel(qh, src3, dst3, pout, srcv, dstv, r0, r1, r2, r3, r4,
                 s0, s1, s2, s3, s4, acc_s):
    c = lax.axis_index("c")
    s = lax.axis_index("s")
    w = c * NS + s
    sl = pl.ds(s * RPT, RPT)

    D = 5
    bufs = (r0, r1, r2, r3, r4)
    sems = (s0, s1, s2, s3, s4)

    def issue(k, t):
        pltpu.async_copy(qh.at[srcv.at[k]], bufs[t], sems[t])

    def waitb(t):
        pltpu.make_async_copy(qh.at[srcv.at[0]], bufs[t], sems[t]).wait()

    def scat(k, t):
        pltpu.sync_copy(bufs[t], acc_s.at[dstv.at[k]], add=True)

    def stage(wi):
        pltpu.sync_copy(src3.at[w, wi], srcv)
        pltpu.sync_copy(dst3.at[w, wi], dstv)

    def ring():
        def group(j, carry2):
            k0 = D * j
            for t in range(D):
                waitb(t)
                issue(k0 + t + D - 1, (t + D - 1) % D)
                scat(k0 + t, t)
            return carry2

        lax.fori_loop(0, (W - D) // D, group, 0)
        b = W - D
        for i in range(D):
            t = (b + i) % D
            waitb(t)
            if b + i + D - 1 < W:
                issue(b + i + D - 1, (b + i + D - 1) % D)
            scat(b + i, t)

    stage(0)
    for t in range(D - 1):
        issue(t, t)
    _zero_slice(r4, acc_s, s, H)
    plsc.subcore_barrier()
    ring()

    def window(wi, carry):
        stage(wi)
        for t in range(D - 1):
            issue(t, t)
        ring()
        return carry

    lax.fori_loop(1, K // W, window, 0)
    plsc.subcore_barrier()
    pltpu.sync_copy(acc_s.at[sl], pout.at[pl.ds(c * NP + s * RPT, RPT)])


DH = H

@functools.partial(
    pl.kernel,
    out_type=jax.ShapeDtypeStruct((NC * NP, DH), jnp.float32),
    mesh=_mesh,
    scratch_types=[
        pltpu.VMEM((W, C), jnp.int32),
        pltpu.VMEM((C, DH), jnp.float32),
        pltpu.VMEM_SHARED((NP, DH), jnp.float32),
    ],
)
def _deg_kernel(dst3, pout, dstv, ones_v, acc_s):
    c = lax.axis_index("c")
    s = lax.axis_index("s")
    w = c * NS + s
    sl = pl.ds(s * RPT, RPT)
    _zero_slice(ones_v, acc_s, s, DH)
    _fill(ones_v, 1.0, DH)
    plsc.subcore_barrier()

    def window(wi, carry):
        pltpu.sync_copy(dst3.at[w, wi], dstv)

        def chunk(k, carry2):
            pltpu.sync_copy(ones_v, acc_s.at[dstv.at[k]], add=True)
            return carry2

        lax.fori_loop(0, W, chunk, 0)
        return carry

    lax.fori_loop(0, K // W, window, 0)
    plsc.subcore_barrier()
    pltpu.sync_copy(acc_s.at[sl], pout.at[pl.ds(c * NP + s * RPT, RPT)])


_BN = 2048
_GN = NP // _BN


def _embedprep_body(h_ref, emb_ref, w0_ref, b0_ref, w1_ref, b1_ref,
                    d0_ref, d1_ref, q_ref, dinv_ref, rdinv_ref):
    hcol = h_ref[...]
    ids = lax.broadcasted_iota(jnp.int32, (_BN, 32), 1)
    oh = (hcol == ids).astype(jnp.float32)
    x = jnp.dot(oh, emb_ref[...], preferred_element_type=jnp.float32)
    x = jnp.maximum(x @ w0_ref[...] + b0_ref[...], 0.0)
    x = x @ w1_ref[...] + b1_ref[...]
    deg = d0_ref[...] + d1_ref[...] + 1.0
    dinv = lax.rsqrt(deg)
    q_ref[...] = dinv * x
    dinv_ref[...] = dinv
    rdinv_ref[...] = jnp.sqrt(deg)


def _embedprep(h2, emb32, W0, b0, W1, b1, d0, d1):
    bs = lambda shape: pl.BlockSpec(shape, lambda i: (i, 0))
    full = lambda shape: pl.BlockSpec(shape, lambda i: (0, 0))
    return pl.pallas_call(
        _embedprep_body,
        grid=(_GN,),
        in_specs=[bs((_BN, 1)), full((32, H)), full((H, H)), full((1, H)),
                  full((H, H)), full((1, H)), bs((_BN, 1)), bs((_BN, 1))],
        out_specs=(bs((_BN, H)), bs((_BN, 1)), bs((_BN, 1))),
        out_shape=(jax.ShapeDtypeStruct((NP, H), jnp.float32),
                   jax.ShapeDtypeStruct((NP, 1), jnp.float32),
                   jax.ShapeDtypeStruct((NP, 1), jnp.float32)),
    )(h2, emb32, W0, b0, W1, b1, d0, d1)


def _node_body(p0_ref, p1_ref, q_ref, dinv_ref, rdinv_ref, sumh_ref,
               steps_ref, cont_ref, acc_ref, hw_ref, hb_ref,
               qn_ref, sumh2_ref, steps2_ref, cont2_ref, acc2_ref):
    q = q_ref[...]
    dinv = dinv_ref[...]
    prop = q * rdinv_ref[...]
    pn = dinv * (p0_ref[...] + p1_ref[...] + q)
    z = jnp.dot(pn, hw_ref[...], preferred_element_type=jnp.float32)
    hp = jax.nn.sigmoid(z + hb_ref[...])
    sumh = sumh_ref[...]
    cont = cont_ref[...]
    pm = (sumh + hp) < 0.99
    pf = pm.astype(jnp.float32) * cont
    steps2 = steps_ref[...] + pf
    sumh2 = sumh + pf * hp
    cond = pm & (steps2 < float(NITER))
    p = jnp.where(cond, sumh2, 1.0 - sumh2)
    acc2_ref[...] = acc_ref[...] + (pn * p + prop * (1.0 - p)) * cont
    qn_ref[...] = dinv * pn
    sumh2_ref[...] = sumh2
    steps2_ref[...] = steps2
    cont2_ref[...] = pf


def _node_step(p0, p1, q, dinv, rdinv, sumh, steps, cont, acc, hw, hb2):
    bs = lambda shape: pl.BlockSpec(shape, lambda i: (i, 0))
    full = lambda shape: pl.BlockSpec(shape, lambda i: (0, 0))
    return pl.pallas_call(
        _node_body,
        grid=(_GN,),
        in_specs=[bs((_BN, H)), bs((_BN, H)), bs((_BN, H)), bs((_BN, 1)),
                  bs((_BN, 1)), bs((_BN, 1)), bs((_BN, 1)), bs((_BN, 1)),
                  bs((_BN, H)), full((H, 1)), full((1, 1))],
        out_specs=(bs((_BN, H)), bs((_BN, 1)), bs((_BN, 1)),
                   bs((_BN, 1)), bs((_BN, H))),
        out_shape=(jax.ShapeDtypeStruct((NP, H), jnp.float32),
                   jax.ShapeDtypeStruct((NP, 1), jnp.float32),
                   jax.ShapeDtypeStruct((NP, 1), jnp.float32),
                   jax.ShapeDtypeStruct((NP, 1), jnp.float32),
                   jax.ShapeDtypeStruct((NP, H), jnp.float32)),
    )(p0, p1, q, dinv, rdinv, sumh, steps, cont, acc, hw, hb2)


def _readout_body(acc_ref, steps_ref, sumh_ref, wr0_ref, br0_ref, wr1_ref,
                  br1_ref, wr2_ref, br2_ref, y_ref, rem_ref, hsum_ref):
    i = pl.program_id(0)

    @pl.when(i == 0)
    def _():
        hsum_ref[...] = jnp.zeros_like(hsum_ref)

    ridx = i * _BN + lax.broadcasted_iota(jnp.int32, (_BN, 1), 0)
    m = (ridx < N).astype(jnp.float32)
    hout = acc_ref[...] / steps_ref[...] * m
    hsum_ref[...] += jnp.sum(hout, axis=0, keepdims=True)
    rem_ref[...] = 1.0 - sumh_ref[...]

    @pl.when(i == _GN - 1)
    def _():
        hg = hsum_ref[...] / float(N)
        y1 = jnp.maximum(hg @ wr0_ref[...] + br0_ref[...], 0.0)
        y2 = jnp.maximum(y1 @ wr1_ref[...] + br1_ref[...], 0.0)
        y_ref[...] = y2 @ wr2_ref[...] + br2_ref[...]


def _readout(acc, steps, sumh, Wr0, br0, Wr1, br1, Wr2, br2):
    bs = lambda shape: pl.BlockSpec(shape, lambda i: (i, 0))
    full = lambda shape: pl.BlockSpec(shape, lambda i: (0, 0))
    return pl.pallas_call(
        _readout_body,
        grid=(_GN,),
        in_specs=[bs((_BN, H)), bs((_BN, 1)), bs((_BN, 1)),
                  full((H, H // 2)), full((1, H // 2)),
                  full((H // 2, H // 4)), full((1, H // 4)),
                  full((H // 4, 1)), full((1, 1))],
        out_specs=(full((1, 1)), bs((_BN, 1))),
        out_shape=(jax.ShapeDtypeStruct((1, 1), jnp.float32),
                   jax.ShapeDtypeStruct((NP, 1), jnp.float32)),
        scratch_shapes=[pltpu.VMEM((1, H), jnp.float32)],
    )(acc, steps, sumh, Wr0, br0, Wr1, br1, Wr2, br2)


def kernel(g, h, e, snorm_n, snorm_e, emb, W0, b0, W1, b1, halt_w, halt_b,
           Wr0, br0, Wr1, br1, Wr2, br2):
    src3 = g[0].reshape(NW, K // W, W, C)
    dst3 = g[1].reshape(NW, K // W, W, C)

    degw = _deg_kernel(dst3)
    d0 = degw[:NP, :1]
    d1 = degw[NP:, :1]

    h2 = jnp.concatenate([h, jnp.zeros((NP - N,), h.dtype)]).reshape(NP, 1)
    emb32 = jnp.concatenate(
        [emb, jnp.zeros((32 - emb.shape[0], H), jnp.float32)], axis=0)
    q, dinv, rdinv = _embedprep(h2, emb32, W0, b0.reshape(1, H),
                                W1, b1.reshape(1, H), d0, d1)

    sumh = jnp.zeros((NP, 1), jnp.float32)
    steps = jnp.ones((NP, 1), jnp.float32)
    cont = jnp.ones((NP, 1), jnp.float32)
    acc = jnp.zeros((NP, H), jnp.float32)
    hb2 = halt_b.reshape(1, 1)

    for _ in range(NITER):
        pcat = _edge_kernel(q, src3, dst3)
        p0, p1 = pcat[:NP], pcat[NP:]
        q, sumh, steps, cont, acc = _node_step(
            p0, p1, q, dinv, rdinv, sumh, steps, cont, acc, halt_w, hb2)

    y, rem = _readout(acc, steps, sumh, Wr0, br0.reshape(1, H // 2),
                      Wr1, br1.reshape(1, H // 4), Wr2, br2.reshape(1, 1))
    return y, steps[:N, 0], rem[:N, 0]


# speedup vs baseline: 1.0100x; 1.0008x over previous
import functools

import jax
import jax.numpy as jnp
from jax import lax
from jax.experimental import pallas as pl
from jax.experimental.pallas import tpu as pltpu
from jax.experimental.pallas import tpu_sc as plsc

N = 10000
NP = 10240
E = 320000
H = 128
NITER = 10
NC = 2
NS = 16
NW = NC * NS
EPW = E // NW
C = 40
K = EPW // C
W = 50
RPT = NP // NS

_mesh = plsc.VectorSubcoreMesh(core_axis_name="c", subcore_axis_name="s")


def _fill(buf, value, width):
    val = jnp.full((16,), value, jnp.float32)

    def row(r, carry):
        for t in range(width // 16):
            buf[r, pl.ds(t * 16, 16)] = val
        return carry

    lax.fori_loop(0, C, row, 0)


def _zero_slice(buf, acc_s, s, width):
    _fill(buf, 0.0, width)

    def cp(i, carry):
        pltpu.sync_copy(buf, acc_s.at[pl.ds(s * RPT + i * C, C)])
        return carry

    lax.fori_loop(0, RPT // C, cp, 0)


@functools.partial(
    pl.kernel,
    out_type=jax.ShapeDtypeStruct((NC * NP, H), jnp.float32),
    mesh=_mesh,
    scratch_types=[
        pltpu.VMEM((W, C), jnp.int32),
        pltpu.VMEM((W, C), jnp.int32),
        pltpu.VMEM((C, H), jnp.float32),
        pltpu.VMEM((C, H), jnp.float32),
        pltpu.VMEM((C, H), jnp.float32),
        pltpu.VMEM((C, H), jnp.float32),
        pltpu.VMEM((C, H), jnp.float32),
        pltpu.SemaphoreType.DMA,
        pltpu.SemaphoreType.DMA,
        pltpu.SemaphoreType.DMA,
        pltpu.SemaphoreType.DMA,
        pltpu.SemaphoreType.DMA,
        pltpu.VMEM_SHARED((NP, H), jnp.float32),
    ],
)
def _edge_kernel(qh, src3, dst3, pout, srcv, dstv, r0, r1, r2, r3, r4,
                 s0, s1, s2, s3, s4, acc_s):
    c = lax.axis_index("c")
    s = lax.axis_index("s")
    w = c * NS + s
    sl = pl.ds(s * RPT, RPT)

    D = 5
    bufs = (r0, r1, r2, r3, r4)
    sems = (s0, s1, s2, s3, s4)

    def issue(k, t):
        pltpu.async_copy(qh.at[srcv.at[k]], bufs[t], sems[t])

    def waitb(t):
        pltpu.make_async_copy(qh.at[srcv.at[0]], bufs[t], sems[t]).wait()

    def scat(k, t):
        pltpu.sync_copy(bufs[t], acc_s.at[dstv.at[k]], add=True)

    def stage(wi):
        pltpu.sync_copy(src3.at[w, wi], srcv)
        pltpu.sync_copy(dst3.at[w, wi], dstv)

    def ring():
        def group(j, carry2):
            k0 = D * j
            for t in range(D):
                waitb(t)
                issue(k0 + t + D - 1, (t + D - 1) % D)
                scat(k0 + t, t)
            return carry2

        lax.fori_loop(0, (W - D) // D, group, 0)
        b = W - D
        for i in range(D):
            t = (b + i) % D
            waitb(t)
            if b + i + D - 1 < W:
                issue(b + i + D - 1, (b + i + D - 1) % D)
            scat(b + i, t)

    stage(0)
    for t in range(D - 1):
        issue(t, t)
    _zero_slice(r4, acc_s, s, H)
    plsc.subcore_barrier()
    ring()

    def window(wi, carry):
        stage(wi)
        for t in range(D - 1):
            issue(t, t)
        ring()
        return carry

    lax.fori_loop(1, K // W, window, 0)
    plsc.subcore_barrier()
    pltpu.sync_copy(acc_s.at[sl], pout.at[pl.ds(c * NP + s * RPT, RPT)])


DH = H

@functools.partial(
    pl.kernel,
    out_type=jax.ShapeDtypeStruct((NC * NP, DH), jnp.float32),
    mesh=_mesh,
    scratch_types=[
        pltpu.VMEM((W, C), jnp.int32),
        pltpu.VMEM((C, DH), jnp.float32),
        pltpu.VMEM_SHARED((NP, DH), jnp.float32),
    ],
)
def _deg_kernel(dst3, pout, dstv, ones_v, acc_s):
    c = lax.axis_index("c")
    s = lax.axis_index("s")
    w = c * NS + s
    sl = pl.ds(s * RPT, RPT)
    _zero_slice(ones_v, acc_s, s, DH)
    _fill(ones_v, 1.0, DH)
    plsc.subcore_barrier()

    def window(wi, carry):
        pltpu.sync_copy(dst3.at[w, wi], dstv)

        def chunk(k, carry2):
            pltpu.sync_copy(ones_v, acc_s.at[dstv.at[k]], add=True)
            return carry2

        lax.fori_loop(0, W, chunk, 0)
        return carry

    lax.fori_loop(0, K // W, window, 0)
    plsc.subcore_barrier()
    pltpu.sync_copy(acc_s.at[sl], pout.at[pl.ds(c * NP + s * RPT, RPT)])


_BN = 2048
_GN = NP // _BN


def _embedprep_body(h_ref, emb_ref, w0_ref, b0_ref, w1_ref, b1_ref,
                    d0_ref, d1_ref, q_ref, dinv_ref, rdinv_ref):
    hcol = h_ref[...]
    ids = lax.broadcasted_iota(jnp.int32, (_BN, 32), 1)
    oh = (hcol == ids).astype(jnp.float32)
    x = jnp.dot(oh, emb_ref[...], preferred_element_type=jnp.float32)
    x = jnp.maximum(x @ w0_ref[...] + b0_ref[...], 0.0)
    x = x @ w1_ref[...] + b1_ref[...]
    deg = d0_ref[...] + d1_ref[...] + 1.0
    dinv = lax.rsqrt(deg)
    q_ref[...] = dinv * x
    dinv_ref[...] = dinv
    rdinv_ref[...] = jnp.sqrt(deg)


def _embedprep(h2, emb32, W0, b0, W1, b1, d0, d1):
    bs = lambda shape: pl.BlockSpec(shape, lambda i: (i, 0))
    full = lambda shape: pl.BlockSpec(shape, lambda i: (0, 0))
    return pl.pallas_call(
        _embedprep_body,
        grid=(_GN,),
        in_specs=[bs((_BN, 1)), full((32, H)), full((H, H)), full((1, H)),
                  full((H, H)), full((1, H)), bs((_BN, 1)), bs((_BN, 1))],
        out_specs=(bs((_BN, H)), bs((_BN, 1)), bs((_BN, 1))),
        out_shape=(jax.ShapeDtypeStruct((NP, H), jnp.float32),
                   jax.ShapeDtypeStruct((NP, 1), jnp.float32),
                   jax.ShapeDtypeStruct((NP, 1), jnp.float32)),
    )(h2, emb32, W0, b0, W1, b1, d0, d1)


def _node_body(p0_ref, p1_ref, q_ref, dinv_ref, rdinv_ref, sumh_ref,
               steps_ref, cont_ref, acc_ref, hw_ref, hb_ref,
               qn_ref, sumh2_ref, steps2_ref, cont2_ref, acc2_ref):
    q = q_ref[...]
    dinv = dinv_ref[...]
    prop = q * rdinv_ref[...]
    pn = dinv * (p0_ref[...] + p1_ref[...] + q)
    z = jnp.dot(pn, hw_ref[...], preferred_element_type=jnp.float32)
    hp = jax.nn.sigmoid(z + hb_ref[...])
    sumh = sumh_ref[...]
    cont = cont_ref[...]
    pm = (sumh + hp) < 0.99
    pf = pm.astype(jnp.float32) * cont
    steps2 = steps_ref[...] + pf
    sumh2 = sumh + pf * hp
    cond = pm & (steps2 < float(NITER))
    p = jnp.where(cond, sumh2, 1.0 - sumh2)
    acc2_ref[...] = acc_ref[...] + (pn * p + prop * (1.0 - p)) * cont
    qn_ref[...] = dinv * pn
    sumh2_ref[...] = sumh2
    steps2_ref[...] = steps2
    cont2_ref[...] = pf


def _node_step(p0, p1, q, dinv, rdinv, sumh, steps, cont, acc, hw, hb2):
    bs = lambda shape: pl.BlockSpec(shape, lambda i: (i, 0))
    full = lambda shape: pl.BlockSpec(shape, lambda i: (0, 0))
    return pl.pallas_call(
        _node_body,
        grid=(_GN,),
        in_specs=[bs((_BN, H)), bs((_BN, H)), bs((_BN, H)), bs((_BN, 1)),
                  bs((_BN, 1)), bs((_BN, 1)), bs((_BN, 1)), bs((_BN, 1)),
                  bs((_BN, H)), full((H, 1)), full((1, 1))],
        out_specs=(bs((_BN, H)), bs((_BN, 1)), bs((_BN, 1)),
                   bs((_BN, 1)), bs((_BN, H))),
        out_shape=(jax.ShapeDtypeStruct((NP, H), jnp.float32),
                   jax.ShapeDtypeStruct((NP, 1), jnp.float32),
                   jax.ShapeDtypeStruct((NP, 1), jnp.float32),
                   jax.ShapeDtypeStruct((NP, 1), jnp.float32),
                   jax.ShapeDtypeStruct((NP, H), jnp.float32)),
    )(p0, p1, q, dinv, rdinv, sumh, steps, cont, acc, hw, hb2)


def _readout_body(acc_ref, steps_ref, sumh_ref, wr0_ref, br0_ref, wr1_ref,
                  br1_ref, wr2_ref, br2_ref, y_ref, rem_ref, hsum_ref):
    i = pl.program_id(0)

    @pl.when(i == 0)
    def _():
        hsum_ref[...] = jnp.zeros_like(hsum_ref)

    ridx = i * _BN + lax.broadcasted_iota(jnp.int32, (_BN, 1), 0)
    m = (ridx < N).astype(jnp.float32)
    hout = acc_ref[...] / steps_ref[...] * m
    hsum_ref[...] += jnp.sum(hout, axis=0, keepdims=True)
    rem_ref[...] = 1.0 - sumh_ref[...]

    @pl.when(i == _GN - 1)
    def _():
        hg = hsum_ref[...] / float(N)
        y1 = jnp.maximum(hg @ wr0_ref[...] + br0_ref[...], 0.0)
        y2 = jnp.maximum(y1 @ wr1_ref[...] + br1_ref[...], 0.0)
        y_ref[...] = y2 @ wr2_ref[...] + br2_ref[...]


def _readout(acc, steps, sumh, Wr0, br0, Wr1, br1, Wr2, br2):
    bs = lambda shape: pl.BlockSpec(shape, lambda i: (i, 0))
    full = lambda shape: pl.BlockSpec(shape, lambda i: (0, 0))
    return pl.pallas_call(
        _readout_body,
        grid=(_GN,),
        in_specs=[bs((_BN, H)), bs((_BN, 1)), bs((_BN, 1)),
                  full((H, H // 2)), full((1, H // 2)),
                  full((H // 2, H // 4)), full((1, H // 4)),
                  full((H // 4, 1)), full((1, 1))],
        out_specs=(full((1, 1)), bs((_BN, 1))),
        out_shape=(jax.ShapeDtypeStruct((1, 1), jnp.float32),
                   jax.ShapeDtypeStruct((NP, 1), jnp.float32)),
        scratch_shapes=[pltpu.VMEM((1, H), jnp.float32)],
    )(acc, steps, sumh, Wr0, br0, Wr1, br1, Wr2, br2)


def kernel(g, h, e, snorm_n, snorm_e, emb, W0, b0, W1, b1, halt_w, halt_b,
           Wr0, br0, Wr1, br1, Wr2, br2):
    src3 = g[0].reshape(NW, K // W, W, C)
    dst3 = g[1].reshape(NW, K // W, W, C)

    degw = _deg_kernel(dst3)
    d0 = degw[:NP, :1]
    d1 = degw[NP:, :1]

    h2 = jnp.concatenate([h, jnp.zeros((NP - N,), h.dtype)]).reshape(NP, 1)
    emb32 = jnp.concatenate(
        [emb, jnp.zeros((32 - emb.shape[0], H), jnp.float32)], axis=0)
    q, dinv, rdinv = _embedprep(h2, emb32, W0, b0.reshape(1, H),
                                W1, b1.reshape(1, H), d0, d1)

    sumh = jnp.zeros((NP, 1), jnp.float32)
    steps = jnp.ones((NP, 1), jnp.float32)
    cont = jnp.ones((NP, 1), jnp.float32)
    acc = jnp.zeros((NP, H), jnp.float32)
    hb2 = halt_b.reshape(1, 1)

    for _ in range(NITER):
        pcat = _edge_kernel(q, src3, dst3)
        p0, p1 = pcat[:NP], pcat[NP:]
        q, sumh, steps, cont, acc = _node_step(
            p0, p1, q, dinv, rdinv, sumh, steps, cont, acc, halt_w, hb2)

    y, rem = _readout(acc, steps, sumh, Wr0, br0.reshape(1, H // 2),
                      Wr1, br1.reshape(1, H // 4), Wr2, br2.reshape(1, 1))
    return y, steps[:N, 0], rem[:N, 0]
